# baseline (speedup 1.0000x reference)
import functools

import jax
import jax.numpy as jnp
from jax import lax
from jax.experimental import pallas as pl
from jax.experimental.pallas import tpu as pltpu
from jax.experimental.pallas import tpu_sc as plsc

E = 16
K = 2
D = 1024
F = 2048
T = 8192
CAP = 2048

BT = 512
EP = 128
BC = 512
NCB = CAP // BC

NW = 32
TPW = T // NW
CH = 64


def _router_body(x_ref, wr_ref, de_ref, do_ref, ge_ref, go_ref,
                 ce_ref, co_ref, cnt_ref, carry_ref):
    i = pl.program_id(0)

    @pl.when(i == 0)
    def _init():
        carry_ref[...] = jnp.zeros_like(carry_ref)

    logits = jnp.dot(x_ref[...], wr_ref[...],
                     preferred_element_type=jnp.float32)
    eidx = lax.broadcasted_iota(jnp.int32, (BT, EP), 1)
    logits = jnp.where(eidx < E, logits, -1e30)
    m = jnp.max(logits, axis=-1, keepdims=True)
    un = jnp.exp(logits - m)
    probs = un / jnp.sum(un, axis=-1, keepdims=True)

    m1 = jnp.max(probs, axis=-1, keepdims=True)
    i1 = jnp.min(jnp.where(probs == m1, eidx, EP), axis=-1, keepdims=True)
    probs2 = jnp.where(eidx == i1, -1.0, probs)
    m2 = jnp.max(probs2, axis=-1, keepdims=True)
    i2 = jnp.min(jnp.where(probs2 == m2, eidx, EP), axis=-1, keepdims=True)
    s = m1 + m2
    p1 = m1 / s
    p2 = m2 / s

    oh1 = (eidx == i1).astype(jnp.float32)
    oh2 = (eidx == i2).astype(jnp.float32)
    oh = oh1 + oh2
    r = lax.broadcasted_iota(jnp.int32, (BT, BT), 0)
    c = lax.broadcasted_iota(jnp.int32, (BT, BT), 1)
    ls = (c < r).astype(jnp.float32)
    excl = jnp.dot(ls, oh, preferred_element_type=jnp.float32,
                   precision=lax.Precision.DEFAULT)
    base = excl + carry_ref[0:1, :]
    pos1 = jnp.sum(jnp.where(eidx == i1, base, 0.0), axis=-1,
                   keepdims=True).astype(jnp.int32)
    pos2 = jnp.sum(jnp.where(eidx == i2, base, 0.0), axis=-1,
                   keepdims=True).astype(jnp.int32)
    total = carry_ref[...] + jnp.broadcast_to(
        jnp.sum(oh, axis=0, keepdims=True), carry_ref.shape)
    carry_ref[...] = total
    cnt_ref[...] = total.astype(jnp.int32)

    v1 = pos1 < CAP
    v2 = pos2 < CAP
    dest1 = i1 * (CAP + 1) + jnp.minimum(pos1, CAP)
    dest2 = i2 * (CAP + 1) + jnp.minimum(pos2, CAP)
    gsrc1 = i1 * CAP + jnp.minimum(pos1, CAP - 1)
    gsrc2 = i2 * CAP + jnp.minimum(pos2, CAP - 1)
    c1 = p1 * v1.astype(jnp.float32)
    c2 = p2 * v2.astype(jnp.float32)

    de_ref[...] = dest1.reshape(1, 1, BT)
    do_ref[...] = dest2.reshape(1, 1, BT)
    ge_ref[...] = gsrc1.reshape(1, 1, BT)
    go_ref[...] = gsrc2.reshape(1, 1, BT)
    ce_ref[...] = c1.reshape(1, 1, BT)
    co_ref[...] = c2.reshape(1, 1, BT)


def _router(x, w_router):
    wr_pad = jnp.pad(w_router, ((0, 0), (0, EP - E)))
    n = T // BT
    i32 = jnp.int32
    outs = pl.pallas_call(
        _router_body,
        grid=(n,),
        in_specs=[
            pl.BlockSpec((BT, D), lambda i: (i, 0)),
            pl.BlockSpec((D, EP), lambda i: (0, 0)),
        ],
        out_specs=[
            pl.BlockSpec((1, 1, BT), lambda i: (i, 0, 0)),
            pl.BlockSpec((1, 1, BT), lambda i: (i, 0, 0)),
            pl.BlockSpec((1, 1, BT), lambda i: (i, 0, 0)),
            pl.BlockSpec((1, 1, BT), lambda i: (i, 0, 0)),
            pl.BlockSpec((1, 1, BT), lambda i: (i, 0, 0)),
            pl.BlockSpec((1, 1, BT), lambda i: (i, 0, 0)),
            pl.BlockSpec((8, EP), lambda i: (0, 0)),
        ],
        out_shape=[
            jax.ShapeDtypeStruct((n, 1, BT), i32),
            jax.ShapeDtypeStruct((n, 1, BT), i32),
            jax.ShapeDtypeStruct((n, 1, BT), i32),
            jax.ShapeDtypeStruct((n, 1, BT), i32),
            jax.ShapeDtypeStruct((n, 1, BT), jnp.float32),
            jax.ShapeDtypeStruct((n, 1, BT), jnp.float32),
            jax.ShapeDtypeStruct((8, EP), i32),
        ],
        scratch_shapes=[pltpu.VMEM((8, EP), jnp.float32)],
    )(x, wr_pad)
    de, do, ge, go, ce, co, cnt = outs
    return (de.reshape(T), do.reshape(T), ge.reshape(T), go.reshape(T),
            ce, co, cnt[0, :E])


def _sc_scatter_body(x_hbm, de_hbm, do_hbm, buf_hbm,
                     ide_v, ido_v, rows_v, sem):
    cc = lax.axis_index("c")
    ss = lax.axis_index("s")
    wid = ss * 2 + cc
    base_t = wid * TPW

    def chunk(ci, carry):
        t0 = base_t + ci * CH
        pltpu.sync_copy(de_hbm.at[pl.ds(t0, CH)], ide_v)
        pltpu.sync_copy(do_hbm.at[pl.ds(t0, CH)], ido_v)
        pltpu.sync_copy(x_hbm.at[pl.ds(t0, CH)], rows_v)
        pltpu.async_copy(rows_v, buf_hbm.at[ide_v], sem).wait()
        pltpu.async_copy(rows_v, buf_hbm.at[ido_v], sem).wait()
        return carry

    lax.fori_loop(0, TPW // CH, chunk, 0)


def _sc_scatter(x, de, do):
    mesh = plsc.VectorSubcoreMesh(core_axis_name="c", subcore_axis_name="s")
    f = pl.kernel(
        _sc_scatter_body,
        mesh=mesh,
        out_type=jax.ShapeDtypeStruct((E * (CAP + 1), D), jnp.float32),
        scratch_types=[
            pltpu.VMEM((CH,), jnp.int32),
            pltpu.VMEM((CH,), jnp.int32),
            pltpu.VMEM((CH, D), jnp.float32),
            pltpu.SemaphoreType.DMA,
        ],
    )
    return f(x, de, do)


def _fc_body(cnt_ref, buf_ref, w1_ref, w2_ref, gelu_ref, y_ref):
    e = pl.program_id(0)
    cb = pl.program_id(1)
    cnt = cnt_ref[e]
    start = cb * BC

    @pl.when(cnt <= start)
    def _zero():
        gelu_ref[...] = jnp.zeros_like(gelu_ref)
        y_ref[...] = jnp.zeros_like(y_ref)

    @pl.when(cnt > start)
    def _compute():
        rows = lax.broadcasted_iota(jnp.int32, (BC, 1), 0) + start
        xb = jnp.where(rows < cnt, buf_ref[0], 0.0)
        h = jnp.dot(xb, w1_ref[0], preferred_element_type=jnp.float32,
                    precision=lax.Precision.DEFAULT)
        g = jax.nn.gelu(h)
        gelu_ref[...] = g[None]
        y_ref[...] = jnp.dot(g, w2_ref[0], preferred_element_type=jnp.float32,
                             precision=lax.Precision.DEFAULT)[None]


def _fc(cnt, buf, w1, w2):
    buf3 = buf.reshape(E, CAP + 1, D)
    grid_spec = pltpu.PrefetchScalarGridSpec(
        num_scalar_prefetch=1,
        grid=(E, NCB),
        in_specs=[
            pl.BlockSpec((1, BC, D), lambda e, cb, cnt: (
                e,
                jnp.minimum(cb, jnp.maximum((cnt[e] + BC - 1) // BC - 1, 0)),
                0)),
            pl.BlockSpec((1, D, F), lambda e, cb, cnt: (e, 0, 0)),
            pl.BlockSpec((1, F, D), lambda e, cb, cnt: (e, 0, 0)),
        ],
        out_specs=[
            pl.BlockSpec((1, BC, F), lambda e, cb, cnt: (e, cb, 0)),
            pl.BlockSpec((1, BC, D), lambda e, cb, cnt: (e, cb, 0)),
        ],
    )
    gelu, y = pl.pallas_call(
        _fc_body,
        grid_spec=grid_spec,
        out_shape=[
            jax.ShapeDtypeStruct((E, CAP, F), jnp.float32),
            jax.ShapeDtypeStruct((E, CAP, D), jnp.float32),
        ],
    )(cnt, buf3, w1, w2)
    return gelu, y


def _sc_gather_body(y_hbm, ge_hbm, go_hbm, se_hbm, so_hbm,
                    idx_v, rows_v, sem):
    cc = lax.axis_index("c")
    ss = lax.axis_index("s")
    wid = ss * 2 + cc
    base_t = wid * TPW

    def chunk(ci, carry):
        t0 = base_t + ci * CH
        pltpu.sync_copy(ge_hbm.at[pl.ds(t0, CH)], idx_v)
        pltpu.async_copy(y_hbm.at[idx_v], rows_v, sem).wait()
        pltpu.sync_copy(rows_v, se_hbm.at[pl.ds(t0, CH)])
        pltpu.sync_copy(go_hbm.at[pl.ds(t0, CH)], idx_v)
        pltpu.async_copy(y_hbm.at[idx_v], rows_v, sem).wait()
        pltpu.sync_copy(rows_v, so_hbm.at[pl.ds(t0, CH)])
        return carry

    lax.fori_loop(0, TPW // CH, chunk, 0)


def _sc_gather(y, ge, go):
    mesh = plsc.VectorSubcoreMesh(core_axis_name="c", subcore_axis_name="s")
    f = pl.kernel(
        _sc_gather_body,
        mesh=mesh,
        out_type=[
            jax.ShapeDtypeStruct((T, D), jnp.float32),
            jax.ShapeDtypeStruct((T, D), jnp.float32),
        ],
        scratch_types=[
            pltpu.VMEM((CH,), jnp.int32),
            pltpu.VMEM((CH, D), jnp.float32),
            pltpu.SemaphoreType.DMA,
        ],
    )
    return f(y, ge, go)


def _combine_body(se_ref, so_ref, ce_ref, co_ref, out_ref):
    ce = jnp.transpose(ce_ref[0])
    co = jnp.transpose(co_ref[0])
    out_ref[...] = ce * se_ref[...] + co * so_ref[...]


def _combine(se, so, ce, co):
    n = T // BT
    return pl.pallas_call(
        _combine_body,
        grid=(n,),
        in_specs=[
            pl.BlockSpec((BT, D), lambda i: (i, 0)),
            pl.BlockSpec((BT, D), lambda i: (i, 0)),
            pl.BlockSpec((1, 1, BT), lambda i: (i, 0, 0)),
            pl.BlockSpec((1, 1, BT), lambda i: (i, 0, 0)),
        ],
        out_specs=pl.BlockSpec((BT, D), lambda i: (i, 0)),
        out_shape=jax.ShapeDtypeStruct((T, D), jnp.float32),
    )(se, so, ce, co)


def kernel(x, w_router, w1, w2):
    de, do, ge, go, ce, co, cnt = _router(x, w_router)
    buf = _sc_scatter(x, de, do)
    gelu, y = _fc(cnt, buf, w1, w2)
    se, so = _sc_gather(y.reshape(E * CAP, D), ge, go)
    out = _combine(se, so, ce, co)
    return out, gelu


# speedup vs baseline: 1.1887x; 1.1887x over previous
import functools

import jax
import jax.numpy as jnp
from jax import lax
from jax.experimental import pallas as pl
from jax.experimental.pallas import tpu as pltpu
from jax.experimental.pallas import tpu_sc as plsc

E = 16
CAPP = 2056
K = 2
D = 1024
F = 2048
T = 8192
CAP = 2048

BT = 512
EP = 128
BC = 512
NCB = CAP // BC

NW = 32
TPW = T // NW
CH = 64


def _router_body(x_ref, wr_ref, de_ref, do_ref, ge_ref, go_ref,
                 ce_ref, co_ref, cnt_ref, carry_ref):
    i = pl.program_id(0)

    @pl.when(i == 0)
    def _init():
        carry_ref[...] = jnp.zeros_like(carry_ref)

    logits = jnp.dot(x_ref[...], wr_ref[...],
                     preferred_element_type=jnp.float32)
    eidx = lax.broadcasted_iota(jnp.int32, (BT, EP), 1)
    logits = jnp.where(eidx < E, logits, -1e30)
    m = jnp.max(logits, axis=-1, keepdims=True)
    un = jnp.exp(logits - m)
    probs = un / jnp.sum(un, axis=-1, keepdims=True)

    m1 = jnp.max(probs, axis=-1, keepdims=True)
    i1 = jnp.min(jnp.where(probs == m1, eidx, EP), axis=-1, keepdims=True)
    probs2 = jnp.where(eidx == i1, -1.0, probs)
    m2 = jnp.max(probs2, axis=-1, keepdims=True)
    i2 = jnp.min(jnp.where(probs2 == m2, eidx, EP), axis=-1, keepdims=True)
    s = m1 + m2
    p1 = m1 / s
    p2 = m2 / s

    oh1 = (eidx == i1).astype(jnp.float32)
    oh2 = (eidx == i2).astype(jnp.float32)
    oh = oh1 + oh2
    r = lax.broadcasted_iota(jnp.int32, (BT, BT), 0)
    c = lax.broadcasted_iota(jnp.int32, (BT, BT), 1)
    ls = (c < r).astype(jnp.float32)
    excl = jnp.dot(ls, oh, preferred_element_type=jnp.float32,
                   precision=lax.Precision.DEFAULT)
    base = excl + carry_ref[0:1, :]
    pos1 = jnp.sum(jnp.where(eidx == i1, base, 0.0), axis=-1,
                   keepdims=True).astype(jnp.int32)
    pos2 = jnp.sum(jnp.where(eidx == i2, base, 0.0), axis=-1,
                   keepdims=True).astype(jnp.int32)
    total = carry_ref[...] + jnp.broadcast_to(
        jnp.sum(oh, axis=0, keepdims=True), carry_ref.shape)
    carry_ref[...] = total
    cnt_ref[...] = total.astype(jnp.int32)

    v1 = pos1 < CAP
    v2 = pos2 < CAP
    dest1 = i1 * CAPP + jnp.minimum(pos1, CAP)
    dest2 = i2 * CAPP + jnp.minimum(pos2, CAP)
    gsrc1 = i1 * CAP + jnp.minimum(pos1, CAP - 1)
    gsrc2 = i2 * CAP + jnp.minimum(pos2, CAP - 1)
    c1 = p1 * v1.astype(jnp.float32)
    c2 = p2 * v2.astype(jnp.float32)

    de_ref[...] = dest1.reshape(1, 1, BT)
    do_ref[...] = dest2.reshape(1, 1, BT)
    ge_ref[...] = gsrc1.reshape(1, 1, BT)
    go_ref[...] = gsrc2.reshape(1, 1, BT)
    ce_ref[...] = c1.reshape(1, 1, BT)
    co_ref[...] = c2.reshape(1, 1, BT)


def _router(x, w_router):
    wr_pad = jnp.pad(w_router, ((0, 0), (0, EP - E)))
    n = T // BT
    i32 = jnp.int32
    outs = pl.pallas_call(
        _router_body,
        grid=(n,),
        in_specs=[
            pl.BlockSpec((BT, D), lambda i: (i, 0)),
            pl.BlockSpec((D, EP), lambda i: (0, 0)),
        ],
        out_specs=[
            pl.BlockSpec((1, 1, BT), lambda i: (i, 0, 0)),
            pl.BlockSpec((1, 1, BT), lambda i: (i, 0, 0)),
            pl.BlockSpec((1, 1, BT), lambda i: (i, 0, 0)),
            pl.BlockSpec((1, 1, BT), lambda i: (i, 0, 0)),
            pl.BlockSpec((1, 1, BT), lambda i: (i, 0, 0)),
            pl.BlockSpec((1, 1, BT), lambda i: (i, 0, 0)),
            pl.BlockSpec((8, EP), lambda i: (0, 0)),
        ],
        out_shape=[
            jax.ShapeDtypeStruct((n, 1, BT), i32),
            jax.ShapeDtypeStruct((n, 1, BT), i32),
            jax.ShapeDtypeStruct((n, 1, BT), i32),
            jax.ShapeDtypeStruct((n, 1, BT), i32),
            jax.ShapeDtypeStruct((n, 1, BT), jnp.float32),
            jax.ShapeDtypeStruct((n, 1, BT), jnp.float32),
            jax.ShapeDtypeStruct((8, EP), i32),
        ],
        scratch_shapes=[pltpu.VMEM((8, EP), jnp.float32)],
    )(x, wr_pad)
    de, do, ge, go, ce, co, cnt = outs
    return (de.reshape(T), do.reshape(T), ge.reshape(T), go.reshape(T),
            ce, co, cnt[0, :E])


def _sc_scatter_body(x_hbm, de_hbm, do_hbm, buf_hbm,
                     ide_v, ido_v, rows_v, sem):
    cc = lax.axis_index("c")
    ss = lax.axis_index("s")
    wid = ss * 2 + cc
    base_t = wid * TPW

    def chunk(ci, carry):
        t0 = base_t + ci * CH
        pltpu.sync_copy(de_hbm.at[pl.ds(t0, CH)], ide_v)
        pltpu.sync_copy(do_hbm.at[pl.ds(t0, CH)], ido_v)
        pltpu.sync_copy(x_hbm.at[pl.ds(t0, CH)], rows_v)
        pltpu.async_copy(rows_v, buf_hbm.at[ide_v], sem).wait()
        pltpu.async_copy(rows_v, buf_hbm.at[ido_v], sem).wait()
        return carry

    lax.fori_loop(0, TPW // CH, chunk, 0)


def _sc_scatter(x, de, do):
    mesh = plsc.VectorSubcoreMesh(core_axis_name="c", subcore_axis_name="s")
    f = pl.kernel(
        _sc_scatter_body,
        mesh=mesh,
        out_type=jax.ShapeDtypeStruct((E * CAPP, D), jnp.float32),
        scratch_types=[
            pltpu.VMEM((CH,), jnp.int32),
            pltpu.VMEM((CH,), jnp.int32),
            pltpu.VMEM((CH, D), jnp.float32),
            pltpu.SemaphoreType.DMA,
        ],
    )
    return f(x, de, do)


def _fc_body(cnt_ref, buf_ref, w1_ref, w2_ref, gelu_ref, y_ref):
    e = pl.program_id(0)
    cb = pl.program_id(1)
    cnt = cnt_ref[e]
    start = cb * BC

    @pl.when(cnt <= start)
    def _zero():
        gelu_ref[...] = jnp.zeros_like(gelu_ref)
        y_ref[...] = jnp.zeros_like(y_ref)

    @pl.when(cnt > start)
    def _compute():
        rows = lax.broadcasted_iota(jnp.int32, (BC, 1), 0) + start
        xb = jnp.where(rows < cnt, buf_ref[0], 0.0)
        h = jnp.dot(xb, w1_ref[0], preferred_element_type=jnp.float32,
                    precision=lax.Precision.DEFAULT)
        g = jax.nn.gelu(h)
        gelu_ref[...] = g[None]
        y_ref[...] = jnp.dot(g, w2_ref[0], preferred_element_type=jnp.float32,
                             precision=lax.Precision.DEFAULT)[None]


def _fc(cnt, buf, w1, w2):
    buf3 = buf.reshape(E, CAPP, D)
    grid_spec = pltpu.PrefetchScalarGridSpec(
        num_scalar_prefetch=1,
        grid=(E, NCB),
        in_specs=[
            pl.BlockSpec((1, BC, D), lambda e, cb, cnt: (
                e,
                jnp.minimum(cb, jnp.maximum((cnt[e] + BC - 1) // BC - 1, 0)),
                0)),
            pl.BlockSpec((1, D, F), lambda e, cb, cnt: (e, 0, 0)),
            pl.BlockSpec((1, F, D), lambda e, cb, cnt: (e, 0, 0)),
        ],
        out_specs=[
            pl.BlockSpec((1, BC, F), lambda e, cb, cnt: (e, cb, 0)),
            pl.BlockSpec((1, BC, D), lambda e, cb, cnt: (e, cb, 0)),
        ],
    )
    gelu, y = pl.pallas_call(
        _fc_body,
        grid_spec=grid_spec,
        out_shape=[
            jax.ShapeDtypeStruct((E, CAP, F), jnp.float32),
            jax.ShapeDtypeStruct((E, CAP, D), jnp.float32),
        ],
    )(cnt, buf3, w1, w2)
    return gelu, y


def _sc_gather_body(y_hbm, ge_hbm, go_hbm, se_hbm, so_hbm,
                    idx_v, rows_v, sem):
    cc = lax.axis_index("c")
    ss = lax.axis_index("s")
    wid = ss * 2 + cc
    base_t = wid * TPW

    def chunk(ci, carry):
        t0 = base_t + ci * CH
        pltpu.sync_copy(ge_hbm.at[pl.ds(t0, CH)], idx_v)
        pltpu.async_copy(y_hbm.at[idx_v], rows_v, sem).wait()
        pltpu.sync_copy(rows_v, se_hbm.at[pl.ds(t0, CH)])
        pltpu.sync_copy(go_hbm.at[pl.ds(t0, CH)], idx_v)
        pltpu.async_copy(y_hbm.at[idx_v], rows_v, sem).wait()
        pltpu.sync_copy(rows_v, so_hbm.at[pl.ds(t0, CH)])
        return carry

    lax.fori_loop(0, TPW // CH, chunk, 0)


def _sc_gather(y, ge, go):
    mesh = plsc.VectorSubcoreMesh(core_axis_name="c", subcore_axis_name="s")
    f = pl.kernel(
        _sc_gather_body,
        mesh=mesh,
        out_type=[
            jax.ShapeDtypeStruct((T, D), jnp.float32),
            jax.ShapeDtypeStruct((T, D), jnp.float32),
        ],
        scratch_types=[
            pltpu.VMEM((CH,), jnp.int32),
            pltpu.VMEM((CH, D), jnp.float32),
            pltpu.SemaphoreType.DMA,
        ],
    )
    return f(y, ge, go)


def _combine_body(se_ref, so_ref, ce_ref, co_ref, out_ref):
    ce = jnp.transpose(ce_ref[0])
    co = jnp.transpose(co_ref[0])
    out_ref[...] = ce * se_ref[...] + co * so_ref[...]


def _combine(se, so, ce, co):
    n = T // BT
    return pl.pallas_call(
        _combine_body,
        grid=(n,),
        in_specs=[
            pl.BlockSpec((BT, D), lambda i: (i, 0)),
            pl.BlockSpec((BT, D), lambda i: (i, 0)),
            pl.BlockSpec((1, 1, BT), lambda i: (i, 0, 0)),
            pl.BlockSpec((1, 1, BT), lambda i: (i, 0, 0)),
        ],
        out_specs=pl.BlockSpec((BT, D), lambda i: (i, 0)),
        out_shape=jax.ShapeDtypeStruct((T, D), jnp.float32),
    )(se, so, ce, co)


def kernel(x, w_router, w1, w2):
    de, do, ge, go, ce, co, cnt = _router(x, w_router)
    buf = _sc_scatter(x, de, do)
    gelu, y = _fc(cnt, buf, w1, w2)
    se, so = _sc_gather(y.reshape(E * CAP, D), ge, go)
    out = _combine(se, so, ce, co)
    return out, gelu
